# baseline (speedup 1.0000x reference)
import functools

import jax
import jax.numpy as jnp
from jax import lax
from jax.experimental import pallas as pl
from jax.experimental.pallas import tpu as pltpu
from jax.experimental.pallas import tpu_sc as plsc

N = 10000
E = 320000
NF = 128
NC = 2
NS = 16
LK = 128
NPAD = 10240
EP = 327680
CPT = EP // (NS * LK)
GK = 16
RPT = NPAD // NS
BT = 1024
GT = (2 * NPAD) // BT


def _mesh():
    return plsc.VectorSubcoreMesh(core_axis_name="c", subcore_axis_name="s",
                                  num_cores=NC, num_subcores=NS)


def _sc_counts(idxs, idxd, ones_row, zeros1d):
    @functools.partial(
        pl.kernel,
        out_type=(jax.ShapeDtypeStruct((NC, NPAD), jnp.float32),
                  jax.ShapeDtypeStruct((NC, NPAD), jnp.float32)),
        mesh=_mesh(),
        scratch_types=[
            pltpu.VMEM((CPT, LK), jnp.int32),
            pltpu.VMEM((LK,), jnp.float32),
            pltpu.VMEM_SHARED((NC * NPAD,), jnp.float32),
            pltpu.VMEM_SHARED((NPAD,), jnp.float32),
        ],
    )
    def k(idxs_hbm, idxd_hbm, ones_hbm, z_hbm, osrc_hbm, odst_hbm,
          idx_v, ones_v, accs, accd):
        c = lax.axis_index("c")
        s = lax.axis_index("s")
        pltpu.sync_copy(ones_hbm, ones_v)
        pltpu.sync_copy(z_hbm.at[pl.ds(s * RPT, RPT)],
                        accd.at[pl.ds(s * RPT, RPT)])
        pltpu.sync_copy(z_hbm.at[pl.ds(s * RPT, RPT)],
                        accs.at[pl.ds(c * NPAD + s * RPT, RPT)])
        pltpu.sync_copy(idxs_hbm.at[c, pl.ds(s * CPT, CPT), :], idx_v)
        plsc.subcore_barrier()

        def body_s(j, carry):
            pltpu.sync_copy(ones_v, accs.at[idx_v.at[j]], add=True)
            return carry
        lax.fori_loop(0, CPT, body_s, 0)

        pltpu.sync_copy(idxd_hbm.at[c, pl.ds(s * CPT, CPT), :], idx_v)

        def body_d(j, carry):
            pltpu.sync_copy(ones_v, accd.at[idx_v.at[j]], add=True)
            return carry
        lax.fori_loop(0, CPT, body_d, 0)

        plsc.subcore_barrier()
        pltpu.sync_copy(accs.at[pl.ds(c * NPAD + s * RPT, RPT)],
                        osrc_hbm.at[c, pl.ds(s * RPT, RPT)])
        pltpu.sync_copy(accd.at[pl.ds(s * RPT, RPT)],
                        odst_hbm.at[c, pl.ds(s * RPT, RPT)])

    return k(idxs, idxd, ones_row, zeros1d)


def _sc_agg(h, idxs, idxd, zrow):
    @functools.partial(
        pl.kernel,
        out_type=jax.ShapeDtypeStruct((NC * NPAD, NF), jnp.float32),
        mesh=_mesh(),
        scratch_types=[
            pltpu.VMEM((2, GK, LK), jnp.int32),
            pltpu.VMEM((2, GK, LK), jnp.int32),
            pltpu.VMEM((LK, NF), jnp.float32),
            pltpu.VMEM((LK, NF), jnp.float32),
            pltpu.VMEM_SHARED((NPAD, NF), jnp.float32),
            pltpu.SemaphoreType.DMA,
            pltpu.SemaphoreType.DMA,
            pltpu.SemaphoreType.DMA,
            pltpu.SemaphoreType.DMA,
            pltpu.SemaphoreType.DMA,
        ],
    )
    def k(h_hbm, idxs_hbm, idxd_hbm, zrow_hbm, out_hbm,
          isrc_v, idst_v, rb0, rb1, acc, sg0, sg1, ss0, ss1, semi):
        c = lax.axis_index("c")
        s = lax.axis_index("s")
        NG = CPT // GK

        pltpu.sync_copy(zrow_hbm, rb0)
        for r in range(RPT // LK):
            pltpu.sync_copy(rb0, acc.at[pl.ds(s * RPT + r * LK, LK), :])
        plsc.subcore_barrier()

        base = s * CPT

        def fire_idx(g, slot):
            a = pltpu.async_copy(
                idxs_hbm.at[c, pl.ds(base + g * GK, GK), :],
                isrc_v.at[slot], semi)
            b = pltpu.async_copy(
                idxd_hbm.at[c, pl.ds(base + g * GK, GK), :],
                idst_v.at[slot], semi)
            return a, b

        def wait_idx(slot):
            pltpu.make_async_copy(
                idxs_hbm.at[c, pl.ds(base, GK), :], isrc_v.at[slot],
                semi).wait()
            pltpu.make_async_copy(
                idxd_hbm.at[c, pl.ds(base, GK), :], idst_v.at[slot],
                semi).wait()

        def fire_gather(slot, row, rb, sem):
            pltpu.async_copy(h_hbm.at[isrc_v.at[slot, row]], rb, sem)

        def wait_dma(rb, sem):
            pltpu.make_async_copy(h_hbm.at[pl.ds(0, LK), :], rb, sem).wait()

        def fire_scatter(slot, row, rb, sem):
            pltpu.async_copy(rb, acc.at[idst_v.at[slot, row]], sem, add=True)

        a, b = fire_idx(0, 0)
        a.wait()
        b.wait()
        if NG > 1:
            fire_idx(1, 1)
        fire_gather(0, 0, rb0, sg0)
        fire_gather(0, 1, rb1, sg1)

        for g in range(NG):
            slot = g % 2

            def pair(t, carry):
                wait_dma(rb0, sg0)
                fire_scatter(slot, 2 * t, rb0, ss0)
                wait_dma(rb1, sg1)
                fire_scatter(slot, 2 * t + 1, rb1, ss1)
                wait_dma(rb0, ss0)
                fire_gather(slot, 2 * t + 2, rb0, sg0)
                wait_dma(rb1, ss1)
                fire_gather(slot, 2 * t + 3, rb1, sg1)
                return carry
            lax.fori_loop(0, GK // 2 - 1, pair, 0)

            wait_dma(rb0, sg0)
            fire_scatter(slot, GK - 2, rb0, ss0)
            wait_dma(rb1, sg1)
            fire_scatter(slot, GK - 1, rb1, ss1)
            wait_dma(rb0, ss0)
            wait_dma(rb1, ss1)
            if g + 1 < NG:
                wait_idx(1 - slot)
                if g + 2 < NG:
                    fire_idx(g + 2, slot)
                fire_gather(1 - slot, 0, rb0, sg0)
                fire_gather(1 - slot, 1, rb1, sg1)

        plsc.subcore_barrier()
        pltpu.sync_copy(acc.at[pl.ds(s * RPT, RPT), :],
                        out_hbm.at[pl.ds(c * NPAD + s * RPT, RPT), :])

    return k(h, idxs, idxd, zrow)


def _rowmask(i, b):
    local = (i * b) % NPAD + lax.broadcasted_iota(jnp.int32, (b, 1), 0)
    return (local < N).astype(jnp.float32)


def _tc_prep(feap, csrc, cdst):
    def body(co_ref, ci_ref, fea_ref, x_ref, ro_ref, ri_ref):
        i = pl.program_id(0)
        mask = _rowmask(i, BT)
        r_out = lax.rsqrt(jnp.maximum(co_ref[...], 1.0)) * mask
        ro_ref[...] = r_out
        ri_ref[...] = lax.rsqrt(jnp.maximum(ci_ref[...], 1.0))
        x_ref[...] = fea_ref[...] * r_out

    return pl.pallas_call(
        body,
        grid=(GT,),
        in_specs=[
            pl.BlockSpec((BT, 1), lambda i: (i, 0)),
            pl.BlockSpec((BT, 1), lambda i: (i, 0)),
            pl.BlockSpec((BT, NF), lambda i: (i, 0)),
        ],
        out_specs=[
            pl.BlockSpec((BT, NF), lambda i: (i, 0)),
            pl.BlockSpec((BT, 1), lambda i: (i, 0)),
            pl.BlockSpec((BT, 1), lambda i: (i, 0)),
        ],
        out_shape=[
            jax.ShapeDtypeStruct((2 * NPAD, NF), jnp.float32),
            jax.ShapeDtypeStruct((2 * NPAD, 1), jnp.float32),
            jax.ShapeDtypeStruct((2 * NPAD, 1), jnp.float32),
        ],
    )(csrc, cdst, feap)


def _tc_layer1(agg, rin, rout, W, b):
    def body(a_ref, ri_ref, ro_ref, w_ref, b_ref, y_ref):
        a = a_ref[...] * ri_ref[...]
        y = jnp.dot(a, w_ref[...], preferred_element_type=jnp.float32) + b_ref[...]
        y_ref[...] = jnp.maximum(y, 0.0) * ro_ref[...]

    return pl.pallas_call(
        body,
        grid=(GT,),
        in_specs=[
            pl.BlockSpec((BT, NF), lambda i: (i, 0)),
            pl.BlockSpec((BT, 1), lambda i: (i, 0)),
            pl.BlockSpec((BT, 1), lambda i: (i, 0)),
            pl.BlockSpec((NF, NF), lambda i: (0, 0)),
            pl.BlockSpec((1, NF), lambda i: (0, 0)),
        ],
        out_specs=pl.BlockSpec((BT, NF), lambda i: (i, 0)),
        out_shape=jax.ShapeDtypeStruct((2 * NPAD, NF), jnp.float32),
    )(agg, rin, rout, W, b.reshape(1, NF))


def _tc_layer2(agg, rin, W, b):
    blocks_per_graph = NPAD // BT

    def body(a_ref, ri_ref, w_ref, b_ref, o_ref):
        i = pl.program_id(0)
        a = a_ref[...] * ri_ref[...]
        h = jnp.dot(a, w_ref[...], preferred_element_type=jnp.float32) + b_ref[...]
        h = jnp.maximum(h, 0.0) * _rowmask(i, BT)
        part = jnp.sum(h, axis=0, keepdims=True)

        @pl.when(i == 0)
        def _():
            o_ref[...] = jnp.zeros_like(o_ref)

        g = i // blocks_per_graph
        sel = (lax.broadcasted_iota(jnp.int32, (2, NF), 0) == g).astype(jnp.float32)
        o_ref[...] += sel * part

    return pl.pallas_call(
        body,
        grid=(GT,),
        in_specs=[
            pl.BlockSpec((BT, NF), lambda i: (i, 0)),
            pl.BlockSpec((BT, 1), lambda i: (i, 0)),
            pl.BlockSpec((NF, NF), lambda i: (0, 0)),
            pl.BlockSpec((1, NF), lambda i: (0, 0)),
        ],
        out_specs=pl.BlockSpec((2, NF), lambda i: (0, 0)),
        out_shape=jax.ShapeDtypeStruct((2, NF), jnp.float32),
    )(agg, rin, W, b.reshape(1, NF))


def _tc_head(hgsum, l1_w, l1_b, l2_w, l2_b, l3_w, l3_b):
    def body(hs_ref, w1_ref, b1_ref, w2_ref, b2_ref, w3_ref, b3_ref,
             l1_ref, l3_ref):
        hg = (hs_ref[0:1, :] * (1.0 / N)) * (hs_ref[1:2, :] * (1.0 / N))
        o1 = jnp.dot(hg, w1_ref[...], preferred_element_type=jnp.float32) + b1_ref[...]
        o2 = jnp.dot(o1, w2_ref[...], preferred_element_type=jnp.float32) + b2_ref[...]
        o3 = jnp.dot(o2, w3_ref[...], preferred_element_type=jnp.float32) + b3_ref[...]
        l1_ref[...] = o1
        l3_ref[...] = o3

    return pl.pallas_call(
        body,
        out_shape=[
            jax.ShapeDtypeStruct((1, 512), jnp.float32),
            jax.ShapeDtypeStruct((1, 2), jnp.float32),
        ],
    )(hgsum, l1_w, l1_b.reshape(1, 512), l2_w, l2_b.reshape(1, 128),
      l3_w, l3_b.reshape(1, 2))


def kernel(fea1, fea2, edge_index1, edge_index2, W1, b1, W2, b2,
           l1_w, l1_b, l2_w, l2_b, l3_w, l3_b):
    f32 = jnp.float32
    pad_rows = jnp.zeros((NPAD - N, NF), f32)
    feap = jnp.concatenate([fea1, pad_rows, fea2, pad_rows], axis=0)

    def prep_idx(ei, g):
        src = jnp.concatenate(
            [ei[0].astype(jnp.int32), jnp.full((EP - E,), N, jnp.int32)])
        dst = jnp.concatenate(
            [ei[1].astype(jnp.int32), jnp.full((EP - E,), N, jnp.int32)])
        return (src + g * NPAD).reshape(NS * CPT, LK), dst.reshape(NS * CPT, LK)

    s1, d1 = prep_idx(edge_index1, 0)
    s2, d2 = prep_idx(edge_index2, 1)
    idxs = jnp.stack([s1, s2])
    idxd = jnp.stack([d1, d2])
    ones_row = jnp.ones((LK,), f32)
    zeros1d = jnp.zeros((NPAD,), f32)
    zrow = jnp.zeros((LK, NF), f32)

    csrc, cdst = _sc_counts(idxs, idxd, ones_row, zeros1d)
    x, rout, rin = _tc_prep(feap, csrc.reshape(-1, 1), cdst.reshape(-1, 1))
    agg1 = _sc_agg(x, idxs, idxd, zrow)
    y = _tc_layer1(agg1, rin, rout, W1, b1)
    agg2 = _sc_agg(y, idxs, idxd, zrow)
    hgsum = _tc_layer2(agg2, rin, W2, b2)
    return _tc_head(hgsum, l1_w, l1_b, l2_w, l2_b, l3_w, l3_b)


# speedup vs baseline: 1.0376x; 1.0376x over previous
import functools

import jax
import jax.numpy as jnp
from jax import lax
from jax.experimental import pallas as pl
from jax.experimental.pallas import tpu as pltpu
from jax.experimental.pallas import tpu_sc as plsc

N = 10000
E = 320000
NF = 128
NC = 2
NS = 16
LK = 128
NPAD = 10240
EP = 327680
CPT = EP // (NS * LK)
GK = 16
RPT = NPAD // NS
BT = 1024
GT = (2 * NPAD) // BT


def _mesh():
    return plsc.VectorSubcoreMesh(core_axis_name="c", subcore_axis_name="s",
                                  num_cores=NC, num_subcores=NS)


def _sc_counts(idxs, idxd, ones_row, zeros1d):
    @functools.partial(
        pl.kernel,
        out_type=(jax.ShapeDtypeStruct((NC, NPAD), jnp.float32),
                  jax.ShapeDtypeStruct((NC, NPAD), jnp.float32)),
        mesh=_mesh(),
        scratch_types=[
            pltpu.VMEM((CPT, LK), jnp.int32),
            pltpu.VMEM((LK,), jnp.float32),
            pltpu.VMEM_SHARED((NC * NPAD,), jnp.float32),
            pltpu.VMEM_SHARED((NPAD,), jnp.float32),
        ],
    )
    def k(idxs_hbm, idxd_hbm, ones_hbm, z_hbm, osrc_hbm, odst_hbm,
          idx_v, ones_v, accs, accd):
        c = lax.axis_index("c")
        s = lax.axis_index("s")
        pltpu.sync_copy(ones_hbm, ones_v)
        pltpu.sync_copy(z_hbm.at[pl.ds(s * RPT, RPT)],
                        accd.at[pl.ds(s * RPT, RPT)])
        pltpu.sync_copy(z_hbm.at[pl.ds(s * RPT, RPT)],
                        accs.at[pl.ds(c * NPAD + s * RPT, RPT)])
        pltpu.sync_copy(idxs_hbm.at[c, pl.ds(s * CPT, CPT), :], idx_v)
        plsc.subcore_barrier()

        def body_s(j, carry):
            pltpu.sync_copy(ones_v, accs.at[idx_v.at[j]], add=True)
            return carry
        lax.fori_loop(0, CPT, body_s, 0)

        pltpu.sync_copy(idxd_hbm.at[c, pl.ds(s * CPT, CPT), :], idx_v)

        def body_d(j, carry):
            pltpu.sync_copy(ones_v, accd.at[idx_v.at[j]], add=True)
            return carry
        lax.fori_loop(0, CPT, body_d, 0)

        plsc.subcore_barrier()
        pltpu.sync_copy(accs.at[pl.ds(c * NPAD + s * RPT, RPT)],
                        osrc_hbm.at[c, pl.ds(s * RPT, RPT)])
        pltpu.sync_copy(accd.at[pl.ds(s * RPT, RPT)],
                        odst_hbm.at[c, pl.ds(s * RPT, RPT)])

    return k(idxs, idxd, ones_row, zeros1d)


def _sc_agg(h, idxs, idxd, zrow):
    @functools.partial(
        pl.kernel,
        out_type=jax.ShapeDtypeStruct((NC * NPAD, NF), jnp.float32),
        mesh=_mesh(),
        scratch_types=[
            pltpu.VMEM((2, GK, LK), jnp.int32),
            pltpu.VMEM((2, GK, LK), jnp.int32),
            pltpu.VMEM((LK, NF), jnp.float32),
            pltpu.VMEM((LK, NF), jnp.float32),
            pltpu.VMEM_SHARED((NPAD, NF), jnp.float32),
            pltpu.SemaphoreType.DMA,
            pltpu.SemaphoreType.DMA,
            pltpu.SemaphoreType.DMA,
        ],
    )
    def k(h_hbm, idxs_hbm, idxd_hbm, zrow_hbm, out_hbm,
          isrc_v, idst_v, rb0, rb1, acc, sg0, sg1, semi):
        c = lax.axis_index("c")
        s = lax.axis_index("s")
        NG = CPT // GK

        pltpu.sync_copy(zrow_hbm, rb0)
        for r in range(RPT // LK):
            pltpu.sync_copy(rb0, acc.at[pl.ds(s * RPT + r * LK, LK), :])
        plsc.subcore_barrier()

        base = s * CPT

        def fire_idx(g, slot):
            a = pltpu.async_copy(
                idxs_hbm.at[c, pl.ds(base + g * GK, GK), :],
                isrc_v.at[slot], semi)
            b = pltpu.async_copy(
                idxd_hbm.at[c, pl.ds(base + g * GK, GK), :],
                idst_v.at[slot], semi)
            return a, b

        def wait_idx(slot):
            pltpu.make_async_copy(
                idxs_hbm.at[c, pl.ds(base, GK), :], isrc_v.at[slot],
                semi).wait()
            pltpu.make_async_copy(
                idxd_hbm.at[c, pl.ds(base, GK), :], idst_v.at[slot],
                semi).wait()

        def fire_gather(slot, row, rb, sem):
            pltpu.async_copy(h_hbm.at[isrc_v.at[slot, row]], rb, sem)

        def wait_dma(rb, sem):
            pltpu.make_async_copy(h_hbm.at[pl.ds(0, LK), :], rb, sem).wait()

        def scatter(slot, row, rb):
            pltpu.sync_copy(rb, acc.at[idst_v.at[slot, row]], add=True)

        a, b = fire_idx(0, 0)
        a.wait()
        b.wait()
        if NG > 1:
            fire_idx(1, 1)
        fire_gather(0, 0, rb0, sg0)

        for g in range(NG):
            slot = g % 2

            def pair(t, carry):
                wait_dma(rb0, sg0)
                fire_gather(slot, 2 * t + 1, rb1, sg1)
                scatter(slot, 2 * t, rb0)
                wait_dma(rb1, sg1)
                fire_gather(slot, 2 * t + 2, rb0, sg0)
                scatter(slot, 2 * t + 1, rb1)
                return carry
            lax.fori_loop(0, GK // 2 - 1, pair, 0)

            wait_dma(rb0, sg0)
            fire_gather(slot, GK - 1, rb1, sg1)
            scatter(slot, GK - 2, rb0)
            wait_dma(rb1, sg1)
            scatter(slot, GK - 1, rb1)
            if g + 1 < NG:
                wait_idx(1 - slot)
                if g + 2 < NG:
                    fire_idx(g + 2, slot)
                fire_gather(1 - slot, 0, rb0, sg0)

        plsc.subcore_barrier()
        pltpu.sync_copy(acc.at[pl.ds(s * RPT, RPT), :],
                        out_hbm.at[pl.ds(c * NPAD + s * RPT, RPT), :])

    return k(h, idxs, idxd, zrow)


def _rowmask(i, b):
    local = (i * b) % NPAD + lax.broadcasted_iota(jnp.int32, (b, 1), 0)
    return (local < N).astype(jnp.float32)


def _tc_prep(feap, csrc, cdst):
    def body(co_ref, ci_ref, fea_ref, x_ref, ro_ref, ri_ref):
        i = pl.program_id(0)
        mask = _rowmask(i, BT)
        r_out = lax.rsqrt(jnp.maximum(co_ref[...], 1.0)) * mask
        ro_ref[...] = r_out
        ri_ref[...] = lax.rsqrt(jnp.maximum(ci_ref[...], 1.0))
        x_ref[...] = fea_ref[...] * r_out

    return pl.pallas_call(
        body,
        grid=(GT,),
        in_specs=[
            pl.BlockSpec((BT, 1), lambda i: (i, 0)),
            pl.BlockSpec((BT, 1), lambda i: (i, 0)),
            pl.BlockSpec((BT, NF), lambda i: (i, 0)),
        ],
        out_specs=[
            pl.BlockSpec((BT, NF), lambda i: (i, 0)),
            pl.BlockSpec((BT, 1), lambda i: (i, 0)),
            pl.BlockSpec((BT, 1), lambda i: (i, 0)),
        ],
        out_shape=[
            jax.ShapeDtypeStruct((2 * NPAD, NF), jnp.float32),
            jax.ShapeDtypeStruct((2 * NPAD, 1), jnp.float32),
            jax.ShapeDtypeStruct((2 * NPAD, 1), jnp.float32),
        ],
    )(csrc, cdst, feap)


def _tc_layer1(agg, rin, rout, W, b):
    def body(a_ref, ri_ref, ro_ref, w_ref, b_ref, y_ref):
        a = a_ref[...] * ri_ref[...]
        y = jnp.dot(a, w_ref[...], preferred_element_type=jnp.float32) + b_ref[...]
        y_ref[...] = jnp.maximum(y, 0.0) * ro_ref[...]

    return pl.pallas_call(
        body,
        grid=(GT,),
        in_specs=[
            pl.BlockSpec((BT, NF), lambda i: (i, 0)),
            pl.BlockSpec((BT, 1), lambda i: (i, 0)),
            pl.BlockSpec((BT, 1), lambda i: (i, 0)),
            pl.BlockSpec((NF, NF), lambda i: (0, 0)),
            pl.BlockSpec((1, NF), lambda i: (0, 0)),
        ],
        out_specs=pl.BlockSpec((BT, NF), lambda i: (i, 0)),
        out_shape=jax.ShapeDtypeStruct((2 * NPAD, NF), jnp.float32),
    )(agg, rin, rout, W, b.reshape(1, NF))


def _tc_layer2(agg, rin, W, b, l1_w, l1_b, l2_w, l2_b, l3_w, l3_b):
    blocks_per_graph = NPAD // BT

    def body(a_ref, ri_ref, w_ref, b_ref, w1_ref, b1_ref, w2_ref, b2_ref,
             w3_ref, b3_ref, l1_ref, l3_ref, o_ref):
        i = pl.program_id(0)
        a = a_ref[...] * ri_ref[...]
        h = jnp.dot(a, w_ref[...], preferred_element_type=jnp.float32) + b_ref[...]
        h = jnp.maximum(h, 0.0) * _rowmask(i, BT)
        part = jnp.sum(h, axis=0, keepdims=True)

        @pl.when(i == 0)
        def _():
            o_ref[...] = jnp.zeros_like(o_ref)

        g = i // blocks_per_graph
        sel = (lax.broadcasted_iota(jnp.int32, (2, NF), 0) == g).astype(jnp.float32)
        o_ref[...] += sel * part

        @pl.when(i == GT - 1)
        def _():
            hg = (o_ref[0:1, :] * (1.0 / N)) * (o_ref[1:2, :] * (1.0 / N))
            o1 = jnp.dot(hg, w1_ref[...],
                         preferred_element_type=jnp.float32) + b1_ref[...]
            o2 = jnp.dot(o1, w2_ref[...],
                         preferred_element_type=jnp.float32) + b2_ref[...]
            o3 = jnp.dot(o2, w3_ref[...],
                         preferred_element_type=jnp.float32) + b3_ref[...]
            l1_ref[...] = o1
            l3_ref[...] = o3

    const = lambda i: (0, 0)
    l1o, l3o, _ = pl.pallas_call(
        body,
        grid=(GT,),
        in_specs=[
            pl.BlockSpec((BT, NF), lambda i: (i, 0)),
            pl.BlockSpec((BT, 1), lambda i: (i, 0)),
            pl.BlockSpec((NF, NF), const),
            pl.BlockSpec((1, NF), const),
            pl.BlockSpec((NF, 512), const),
            pl.BlockSpec((1, 512), const),
            pl.BlockSpec((512, NF), const),
            pl.BlockSpec((1, NF), const),
            pl.BlockSpec((NF, 2), const),
            pl.BlockSpec((1, 2), const),
        ],
        out_specs=[
            pl.BlockSpec((1, 512), const),
            pl.BlockSpec((1, 2), const),
            pl.BlockSpec((2, NF), const),
        ],
        out_shape=[
            jax.ShapeDtypeStruct((1, 512), jnp.float32),
            jax.ShapeDtypeStruct((1, 2), jnp.float32),
            jax.ShapeDtypeStruct((2, NF), jnp.float32),
        ],
    )(agg, rin, W, b.reshape(1, NF), l1_w, l1_b.reshape(1, 512),
      l2_w, l2_b.reshape(1, NF), l3_w, l3_b.reshape(1, 2))
    return l1o, l3o


def kernel(fea1, fea2, edge_index1, edge_index2, W1, b1, W2, b2,
           l1_w, l1_b, l2_w, l2_b, l3_w, l3_b):
    f32 = jnp.float32
    pad_rows = jnp.zeros((NPAD - N, NF), f32)
    feap = jnp.concatenate([fea1, pad_rows, fea2, pad_rows], axis=0)

    def prep_idx(ei, g):
        src = jnp.concatenate(
            [ei[0].astype(jnp.int32), jnp.full((EP - E,), N, jnp.int32)])
        dst = jnp.concatenate(
            [ei[1].astype(jnp.int32), jnp.full((EP - E,), N, jnp.int32)])
        return (src + g * NPAD).reshape(NS * CPT, LK), dst.reshape(NS * CPT, LK)

    s1, d1 = prep_idx(edge_index1, 0)
    s2, d2 = prep_idx(edge_index2, 1)
    idxs = jnp.stack([s1, s2])
    idxd = jnp.stack([d1, d2])
    ones_row = jnp.ones((LK,), f32)
    zeros1d = jnp.zeros((NPAD,), f32)
    zrow = jnp.zeros((LK, NF), f32)

    csrc, cdst = _sc_counts(idxs, idxd, ones_row, zeros1d)
    x, rout, rin = _tc_prep(feap, csrc.reshape(-1, 1), cdst.reshape(-1, 1))
    agg1 = _sc_agg(x, idxs, idxd, zrow)
    y = _tc_layer1(agg1, rin, rout, W1, b1)
    agg2 = _sc_agg(y, idxs, idxd, zrow)
    return _tc_layer2(agg2, rin, W2, b2,
                      l1_w, l1_b, l2_w, l2_b, l3_w, l3_b)
